# Initial kernel scaffold; baseline (speedup 1.0000x reference)
import jax
import jax.numpy as jnp
from jax.experimental import pallas as pl


def kernel(inputs, token_table, pos_table):
    raise NotImplementedError("write your pallas kernel here")


# speedup vs baseline: 3.3398x; 3.3398x over previous
import functools

import jax
import jax.numpy as jnp
from jax import lax
from jax.experimental import pallas as pl
from jax.experimental.pallas import tpu as pltpu
from jax.experimental.pallas import tpu_sc as plsc

_VOCAB = 100000
_SEQ = 200
_EMBED = 64
_BATCH = 4096

_info = plsc.get_sparse_core_info()
_NC, _NS, _L = _info.num_cores, _info.num_subcores, _info.num_lanes
_NW = _NC * _NS

_ROWS = _BATCH * _SEQ
_RPW = _ROWS // _NW
_SEQ_PER_CHUNK = 2
_CHUNK = _SEQ_PER_CHUNK * _SEQ
_NCHUNKS = _RPW // _CHUNK
_IDXW = 100
_GPC = _CHUNK // _IDXW


def _build():
  mesh = plsc.VectorSubcoreMesh(core_axis_name="c", subcore_axis_name="s")

  @functools.partial(
      pl.kernel,
      mesh=mesh,
      compiler_params=pltpu.CompilerParams(use_tc_tiling_on_sc=False),
      out_type=jax.ShapeDtypeStruct((_ROWS, _EMBED), jnp.float32),
      scratch_types=[
          pltpu.VMEM((_GPC, _IDXW), jnp.int32),
          pltpu.VMEM((_CHUNK, _EMBED), jnp.float32),
          pltpu.VMEM((_CHUNK, _EMBED), jnp.float32),
          pltpu.SemaphoreType.DMA,
      ],
  )
  def k(idx_hbm, table_hbm, pos_hbm, out_hbm, idx_v, rows_v, pos_v, sem):
    wid = lax.axis_index("s") * _NC + lax.axis_index("c")

    for s in range(_SEQ_PER_CHUNK):
      pltpu.sync_copy(pos_hbm, pos_v.at[pl.ds(s * _SEQ, _SEQ)])

    def chunk_body(c, carry):
      base = wid * _RPW + c * _CHUNK
      idx_row0 = wid * (_RPW // _IDXW) + c * _GPC
      pltpu.sync_copy(idx_hbm.at[pl.ds(idx_row0, _GPC)], idx_v)
      cps = [
          pltpu.async_copy(
              table_hbm.at[idx_v.at[g]],
              rows_v.at[pl.ds(g * _IDXW, _IDXW)],
              sem,
          )
          for g in range(_GPC)
      ]
      for cp in cps:
        cp.wait()

      def add_body(i, carry2):
        for j in range(_EMBED // _L):
          sl = pl.ds(j * _L, _L)
          rows_v[i, sl] = rows_v[i, sl] + pos_v[i, sl]
        return carry2

      lax.fori_loop(0, _CHUNK, add_body, 0)
      pltpu.sync_copy(rows_v, out_hbm.at[pl.ds(base, _CHUNK)])
      return carry

    lax.fori_loop(0, _NCHUNKS, chunk_body, 0)

  return k


_kernel_call = _build()


@jax.jit
def kernel(inputs, token_table, pos_table):
  idx = inputs.astype(jnp.int32).reshape(_ROWS // _IDXW, _IDXW)
  out = _kernel_call(idx, token_table, pos_table)
  return out.reshape(_BATCH, _SEQ, _EMBED)


# speedup vs baseline: 4.2201x; 1.2636x over previous
import functools

import jax
import jax.numpy as jnp
from jax import lax
from jax.experimental import pallas as pl
from jax.experimental.pallas import tpu as pltpu
from jax.experimental.pallas import tpu_sc as plsc

_VOCAB = 100000
_SEQ = 200
_EMBED = 64
_BATCH = 4096

_info = plsc.get_sparse_core_info()
_NC, _NS, _L = _info.num_cores, _info.num_subcores, _info.num_lanes
_NW = _NC * _NS

_ROWS = _BATCH * _SEQ
_RPW = _ROWS // _NW
_CHUNK = _SEQ
_NCHUNKS = _RPW // _CHUNK
_NB = 4
_IDXW = 100
_GPC = _CHUNK // _IDXW
_IROWS = _RPW // _IDXW


def _build():
  mesh = plsc.VectorSubcoreMesh(core_axis_name="c", subcore_axis_name="s")

  @functools.partial(
      pl.kernel,
      mesh=mesh,
      compiler_params=pltpu.CompilerParams(use_tc_tiling_on_sc=False),
      out_type=jax.ShapeDtypeStruct((_ROWS, _EMBED), jnp.float32),
      scratch_types=[
          pltpu.VMEM((_IROWS, _IDXW), jnp.int32),
          pltpu.VMEM((_CHUNK, _EMBED), jnp.float32),
      ]
      + [pltpu.VMEM((_CHUNK, _EMBED), jnp.float32) for _ in range(_NB)]
      + [pltpu.SemaphoreType.DMA for _ in range(2 * _NB)],
  )
  def k(idx_hbm, table_hbm, pos_hbm, out_hbm, idx_all, pos_v, *bufs):
    rows = bufs[:_NB]
    gsem = bufs[_NB:2 * _NB]
    ssem = bufs[2 * _NB:]
    wid = lax.axis_index("s") * _NC + lax.axis_index("c")

    pltpu.sync_copy(idx_hbm.at[pl.ds(wid * _IROWS, _IROWS)], idx_all)
    pltpu.sync_copy(pos_hbm, pos_v)

    def fire(n, b):
      for g in range(_GPC):
        pltpu.async_copy(
            table_hbm.at[idx_all.at[n * _GPC + g]],
            rows[b].at[pl.ds(g * _IDXW, _IDXW)],
            gsem[b],
        )

    def wait_gather(b):
      pltpu.make_async_copy(
          out_hbm.at[pl.ds(0, _CHUNK)], rows[b], gsem[b]).wait()

    def wait_store(b):
      pltpu.make_async_copy(
          rows[b], out_hbm.at[pl.ds(0, _CHUNK)], ssem[b]).wait()

    def add_pos(buf):
      def body(i, carry):
        for j in range(_EMBED // _L):
          sl = pl.ds(j * _L, _L)
          plsc.addupdate(buf.at[i, sl], pos_v[i, sl])
        return carry

      lax.fori_loop(0, _CHUNK, body, 0, unroll=2)

    def step(c, bc, do_fire, do_store_wait):
      bw = (bc + 2) % _NB
      if do_store_wait:
        wait_store(bw)
      if do_fire:
        fire(c + 2, bw)
      wait_gather(bc)
      add_pos(rows[bc])
      pltpu.async_copy(
          rows[bc], out_hbm.at[pl.ds(wid * _RPW + c * _CHUNK, _CHUNK)],
          ssem[bc])

    fire(0, 0)
    fire(1, 1)
    step(0, 0, True, False)
    step(1, 1, True, False)

    def group(cc, carry):
      for b in range(_NB):
        step(cc * _NB + 2 + b, (2 + b) % _NB, True, True)
      return carry

    lax.fori_loop(0, (_NCHUNKS - 4) // _NB, group, 0)

    step(_NCHUNKS - 2, (_NCHUNKS - 2) % _NB, False, True)
    step(_NCHUNKS - 1, (_NCHUNKS - 1) % _NB, False, True)
    wait_store((_NCHUNKS - 2) % _NB)
    wait_store((_NCHUNKS - 1) % _NB)

  return k


_kernel_call = _build()


@jax.jit
def kernel(inputs, token_table, pos_table):
  idx = inputs.astype(jnp.int32).reshape(_ROWS // _IDXW, _IDXW)
  out = _kernel_call(idx, token_table, pos_table)
  return out.reshape(_BATCH, _SEQ, _EMBED)
